# baseline (speedup 1.0000x reference)
import functools

import jax
import jax.numpy as jnp
from jax import lax
from jax.experimental import pallas as pl
from jax.experimental.pallas import tpu as pltpu
from jax.experimental.pallas import tpu_sc as plsc

NUM_EMB = 100000
D = 128
B = 4096

_info = plsc.get_sparse_core_info()
_NC, _NS = _info.num_cores, _info.num_subcores
_NW = _NC * _NS
_BPW = B // _NW

_sc_mesh = plsc.VectorSubcoreMesh(core_axis_name="c", subcore_axis_name="s")


@functools.partial(
    pl.kernel,
    mesh=_sc_mesh,
    out_type=[
        jax.ShapeDtypeStruct((B, D), jnp.float32),
        jax.ShapeDtypeStruct((B, D), jnp.float32),
    ],
    scratch_types=[
        pltpu.VMEM((_BPW,), jnp.int32),
        pltpu.VMEM((_BPW,), jnp.int32),
        pltpu.VMEM((_BPW, D), jnp.float32),
        pltpu.VMEM((_BPW, D), jnp.float32),
        pltpu.SemaphoreType.DMA,
        pltpu.SemaphoreType.DMA,
        pltpu.SemaphoreType.DMA,
        pltpu.SemaphoreType.DMA,
    ],
)
def _sc_gather(cur_hbm, last_hbm, table_hbm, out1_hbm, out2_hbm,
               idx1_v, idx2_v, rows1_v, rows2_v, sem1, sem2, sem3, sem4):
    wid = lax.axis_index("s") * _NC + lax.axis_index("c")
    base = wid * _BPW
    i1 = pltpu.async_copy(cur_hbm.at[pl.ds(base, _BPW)], idx1_v, sem3)
    i2 = pltpu.async_copy(last_hbm.at[pl.ds(base, _BPW)], idx2_v, sem4)
    i1.wait()
    c1 = pltpu.async_copy(table_hbm.at[idx1_v], rows1_v, sem1)
    i2.wait()
    c2 = pltpu.async_copy(table_hbm.at[idx2_v], rows2_v, sem2)
    c1.wait()
    w1 = pltpu.async_copy(rows1_v, out1_hbm.at[pl.ds(base, _BPW)], sem3)
    c2.wait()
    w2 = pltpu.async_copy(rows2_v, out2_hbm.at[pl.ds(base, _BPW)], sem4)
    w1.wait()
    w2.wait()


def _proj_body(cur_ref, last_ref, w_ref, b_ref, o_ref):
    w1 = w_ref[:, :D]
    w2 = w_ref[:, D:]
    o_ref[...] = (
        lax.dot_general(cur_ref[...], w1, (((1,), (1,)), ((), ())),
                        preferred_element_type=jnp.float32)
        + lax.dot_general(last_ref[...], w2, (((1,), (1,)), ((), ())),
                          preferred_element_type=jnp.float32)
        + b_ref[...]
    )


_BM = 2048


@jax.jit
def _project(cur_rows, last_rows, W, b2d):
    return pl.pallas_call(
        _proj_body,
        grid=(B // _BM,),
        in_specs=[
            pl.BlockSpec((_BM, D), lambda i: (i, 0)),
            pl.BlockSpec((_BM, D), lambda i: (i, 0)),
            pl.BlockSpec((D, 2 * D), lambda i: (0, 0)),
            pl.BlockSpec((1, D), lambda i: (0, 0)),
        ],
        out_specs=pl.BlockSpec((_BM, D), lambda i: (i, 0)),
        out_shape=jax.ShapeDtypeStruct((B, D), jnp.float32),
    )(cur_rows, last_rows, W, b2d)


def kernel(current_node, actionList, table, W, b):
    cur_rows, last_rows = _sc_gather(
        current_node.astype(jnp.int32), actionList.astype(jnp.int32), table)
    return _project(cur_rows, last_rows, W, b.reshape(1, D))


# speedup vs baseline: 1.0016x; 1.0016x over previous
import functools

import jax
import jax.numpy as jnp
from jax import lax
from jax.experimental import pallas as pl
from jax.experimental.pallas import tpu as pltpu
from jax.experimental.pallas import tpu_sc as plsc

NUM_EMB = 100000
D = 128
B = 4096

_info = plsc.get_sparse_core_info()
_NC, _NS = _info.num_cores, _info.num_subcores
_NW = _NC * _NS
_BPW = B // _NW

_sc_mesh = plsc.VectorSubcoreMesh(core_axis_name="c", subcore_axis_name="s")


@functools.partial(
    pl.kernel,
    mesh=_sc_mesh,
    out_type=[
        jax.ShapeDtypeStruct((B, D), jnp.float32),
        jax.ShapeDtypeStruct((B, D), jnp.float32),
    ],
    scratch_types=[
        pltpu.VMEM((_BPW,), jnp.int32),
        pltpu.VMEM((_BPW,), jnp.int32),
        pltpu.VMEM((_BPW, D), jnp.float32),
        pltpu.VMEM((_BPW, D), jnp.float32),
        pltpu.SemaphoreType.DMA,
        pltpu.SemaphoreType.DMA,
        pltpu.SemaphoreType.DMA,
        pltpu.SemaphoreType.DMA,
    ],
)
def _sc_gather(cur_hbm, last_hbm, table_hbm, out1_hbm, out2_hbm,
               idx1_v, idx2_v, rows1_v, rows2_v, sem1, sem2, sem3, sem4):
    wid = lax.axis_index("s") * _NC + lax.axis_index("c")
    base = wid * _BPW
    i1 = pltpu.async_copy(cur_hbm.at[pl.ds(base, _BPW)], idx1_v, sem3)
    i2 = pltpu.async_copy(last_hbm.at[pl.ds(base, _BPW)], idx2_v, sem4)
    i1.wait()
    c1 = pltpu.async_copy(table_hbm.at[idx1_v], rows1_v, sem1)
    i2.wait()
    c2 = pltpu.async_copy(table_hbm.at[idx2_v], rows2_v, sem2)
    c1.wait()
    w1 = pltpu.async_copy(rows1_v, out1_hbm.at[pl.ds(base, _BPW)], sem3)
    c2.wait()
    w2 = pltpu.async_copy(rows2_v, out2_hbm.at[pl.ds(base, _BPW)], sem4)
    w1.wait()
    w2.wait()


def _proj_body(cur_ref, last_ref, wt_ref, b_ref, o_ref):
    w1 = wt_ref[:D, :]
    w2 = wt_ref[D:, :]
    o_ref[...] = (
        lax.dot_general(cur_ref[...], w1, (((1,), (0,)), ((), ())),
                        preferred_element_type=jnp.float32)
        + lax.dot_general(last_ref[...], w2, (((1,), (0,)), ((), ())),
                          preferred_element_type=jnp.float32)
        + b_ref[...]
    )


_BM = 2048


@jax.jit
def _project(cur_rows, last_rows, W, b2d):
    return pl.pallas_call(
        _proj_body,
        grid=(B // _BM,),
        in_specs=[
            pl.BlockSpec((_BM, D), lambda i: (i, 0)),
            pl.BlockSpec((_BM, D), lambda i: (i, 0)),
            pl.BlockSpec((2 * D, D), lambda i: (0, 0)),
            pl.BlockSpec((1, D), lambda i: (0, 0)),
        ],
        out_specs=pl.BlockSpec((_BM, D), lambda i: (i, 0)),
        out_shape=jax.ShapeDtypeStruct((B, D), jnp.float32),
    )(cur_rows, last_rows, W.T, b2d)


def kernel(current_node, actionList, table, W, b):
    cur_rows, last_rows = _sc_gather(
        current_node.astype(jnp.int32), actionList.astype(jnp.int32), table)
    return _project(cur_rows, last_rows, W, b.reshape(1, D))
